# Initial kernel scaffold; baseline (speedup 1.0000x reference)
import jax
import jax.numpy as jnp
from jax.experimental import pallas as pl


def kernel(X, edge_index, edge_values, gpr_weights):
    raise NotImplementedError("write your pallas kernel here")


# speedup vs baseline: 1.9906x; 1.9906x over previous
import functools

import jax
import jax.numpy as jnp
from jax import lax
from jax.experimental import pallas as pl
from jax.experimental.pallas import tpu as pltpu
from jax.experimental.pallas import tpu_sc as plsc

_HOPS = 10
_NC = 2
_NS = 16
_LANES = 16
_CHUNK = 128


@functools.lru_cache(maxsize=None)
def _build(n, d, ep):
    dh = d // _NC
    nvec = dh // _LANES
    n_pad = -(-n // (_NS * _CHUNK)) * (_NS * _CHUNK)
    rows_tile = n_pad // _NS
    nk = rows_tile // _CHUNK
    cpt = ep // (_NS * _CHUNK)

    mesh = plsc.VectorSubcoreMesh(core_axis_name="c", subcore_axis_name="s")

    grp = 8
    ntg = cpt // grp

    def body(xr, col2, row2, ev2, gam, h, outr,
             col8_v, row8_v, ev8_v, rows_v, stage_v, zbuf, out_acc, gam_v,
             acc, sem):
        c = lax.axis_index("c")
        s = lax.axis_index("s")
        cn = c * n_pad
        ra = s * rows_tile
        eb = s * cpt

        pltpu.sync_copy(gam, gam_v)

        zv = jnp.zeros((_LANES,), jnp.float32)
        def zr(r, carry):
            for jf in range(nvec):
                zbuf[r, pl.ds(jf * _LANES, _LANES)] = zv
            return carry
        lax.fori_loop(0, _CHUNK, zr, 0)

        gv = gam_v[...]
        g0 = gv[0]
        for k in range(nk):
            pltpu.sync_copy(xr.at[pl.ds(cn + ra + k * _CHUNK, _CHUNK)],
                            stage_v)
            def initk(r, carry, _k=k):
                for jf in range(nvec):
                    sl = pl.ds(jf * _LANES, _LANES)
                    out_acc[_k * _CHUNK + r, sl] = g0 * stage_v[r, sl]
                return carry
            lax.fori_loop(0, _CHUNK, initk, 0)

        def zero_acc():
            for k in range(nk):
                pltpu.sync_copy(zbuf, acc.at[pl.ds(ra + k * _CHUNK, _CHUNK)])

        def edge_phase(src):
            def group(gi, carry):
                gb = eb + gi * grp
                pltpu.sync_copy(col2.at[pl.ds(gb, grp)], col8_v)
                pltpu.sync_copy(row2.at[pl.ds(gb, grp)], row8_v)
                pltpu.sync_copy(ev2.at[pl.ds(gb, grp)], ev8_v)

                def adjj(j, c2):
                    for m in range(_CHUNK // _LANES):
                        sl = pl.ds(m * _LANES, _LANES)
                        col8_v[j, sl] = col8_v[j, sl] + cn
                    return c2
                lax.fori_loop(0, grp, adjj, 0)

                def chunk(j, c2):
                    pltpu.async_copy(src.at[col8_v.at[j]], rows_v, sem).wait()
                    def scale(g, icarry):
                        evv = ev8_v[j, pl.ds(g * _LANES, _LANES)]
                        for lane in range(_LANES):
                            ev = evv[lane]
                            e = g * _LANES + lane
                            for jf in range(nvec):
                                sl = pl.ds(jf * _LANES, _LANES)
                                rows_v[e, sl] = ev * rows_v[e, sl]
                        return icarry
                    lax.fori_loop(0, _CHUNK // _LANES, scale, 0)
                    pltpu.sync_copy(rows_v, acc.at[row8_v.at[j]], add=True)
                    return c2
                lax.fori_loop(0, grp, chunk, 0)
                return carry
            lax.fori_loop(0, ntg, group, 0)

        def writeback(l, write_h):
            g = gv[l]
            for k in range(nk):
                pltpu.sync_copy(acc.at[pl.ds(ra + k * _CHUNK, _CHUNK)],
                                stage_v)
                if write_h:
                    pltpu.sync_copy(
                        stage_v, h.at[pl.ds(cn + ra + k * _CHUNK, _CHUNK)])
                def wb(r, carry, _k=k):
                    for jf in range(nvec):
                        sl = pl.ds(jf * _LANES, _LANES)
                        out_acc[_k * _CHUNK + r, sl] = (
                            out_acc[_k * _CHUNK + r, sl] + g * stage_v[r, sl])
                    return carry
                lax.fori_loop(0, _CHUNK, wb, 0)

        for l in range(1, _HOPS + 1):
            zero_acc()
            plsc.subcore_barrier()
            edge_phase(xr if l == 1 else h)
            plsc.subcore_barrier()
            writeback(l, l < _HOPS)

        pltpu.sync_copy(out_acc, outr.at[pl.ds(cn + ra, rows_tile)])

    f32 = jnp.float32
    i32 = jnp.int32
    return pl.kernel(
        body,
        out_type=(
            jax.ShapeDtypeStruct((_NC * n_pad, dh), f32),
            jax.ShapeDtypeStruct((_NC * n_pad, dh), f32),
        ),
        mesh=mesh,
        compiler_params=pltpu.CompilerParams(use_tc_tiling_on_sc=False),
        scratch_types=[
            pltpu.VMEM((grp, _CHUNK), i32),
            pltpu.VMEM((grp, _CHUNK), i32),
            pltpu.VMEM((grp, _CHUNK), f32),
            pltpu.VMEM((_CHUNK, dh), f32),
            pltpu.VMEM((_CHUNK, dh), f32),
            pltpu.VMEM((_CHUNK, dh), f32),
            pltpu.VMEM((rows_tile, dh), f32),
            pltpu.VMEM((_LANES,), f32),
            pltpu.VMEM_SHARED((n_pad, dh), f32),
            pltpu.SemaphoreType.DMA,
        ],
    )


def kernel(X, edge_index, edge_values, gpr_weights):
    n, d = X.shape
    e = edge_values.shape[0]
    dh = d // _NC
    ep = -(-e // (_NS * _CHUNK * 8)) * (_NS * _CHUNK * 8)
    n_pad = -(-n // (_NS * _CHUNK)) * (_NS * _CHUNK)

    row = jnp.concatenate(
        [edge_index[0], jnp.zeros((ep - e,), jnp.int32)]).reshape(-1, _CHUNK)
    col = jnp.concatenate(
        [edge_index[1], jnp.zeros((ep - e,), jnp.int32)]).reshape(-1, _CHUNK)
    ev = jnp.concatenate(
        [edge_values, jnp.zeros((ep - e,), jnp.float32)]).reshape(-1, _CHUNK)
    xr = jnp.pad(X.reshape(n, _NC, dh).transpose(1, 0, 2),
                 ((0, 0), (0, n_pad - n), (0, 0))).reshape(_NC * n_pad, dh)
    gam = jnp.zeros((_LANES,), jnp.float32).at[:gpr_weights.shape[0]].set(
        gpr_weights)

    _, outr = _build(n, d, ep)(xr, col, row, ev, gam)
    return outr.reshape(_NC, n_pad, dh)[:, :n].transpose(1, 0, 2).reshape(n, d)


# speedup vs baseline: 4.1789x; 2.0993x over previous
import functools

import jax
import jax.numpy as jnp
from jax import lax
from jax.experimental import pallas as pl
from jax.experimental.pallas import tpu as pltpu
from jax.experimental.pallas import tpu_sc as plsc

_HOPS = 10
_NC = 2
_NS = 16
_LANES = 16
_CHUNK = 128


@functools.lru_cache(maxsize=None)
def _build(n, d, ep):
    dh = d // _NC
    nvec = dh // _LANES
    n_pad = -(-n // (_NS * _CHUNK)) * (_NS * _CHUNK)
    rows_tile = n_pad // _NS
    nk = rows_tile // _CHUNK
    cpt = ep // (_NS * _CHUNK)

    mesh = plsc.VectorSubcoreMesh(core_axis_name="c", subcore_axis_name="s")

    grp = 16
    ntg = cpt // grp

    def body(xr, col2, row2, ev2, gam, h, outr,
             col8_v, row8_v, ev8_v, rows2, stage_v, zbuf, out_acc,
             gam_v, acc, sems):
        c = lax.axis_index("c")
        s = lax.axis_index("s")
        cn = c * n_pad
        ra = s * rows_tile
        eb = s * cpt

        pltpu.sync_copy(gam, gam_v)

        zv = jnp.zeros((_LANES,), jnp.float32)
        def zr(r, carry):
            for jf in range(nvec):
                zbuf[r, pl.ds(jf * _LANES, _LANES)] = zv
            return carry
        lax.fori_loop(0, _CHUNK, zr, 0)

        g0 = gam_v[0, :]
        for k in range(nk):
            pltpu.sync_copy(xr.at[pl.ds(cn + ra + k * _CHUNK, _CHUNK)],
                            stage_v)
            def initk(r, carry, _k=k):
                for jf in range(nvec):
                    sl = pl.ds(jf * _LANES, _LANES)
                    out_acc[_k * _CHUNK + r, sl] = g0 * stage_v[r, sl]
                return carry
            lax.fori_loop(0, _CHUNK, initk, 0)

        def zero_acc():
            for k in range(nk):
                pltpu.sync_copy(zbuf, acc.at[pl.ds(ra + k * _CHUNK, _CHUNK)])

        def edge_phase(src):
            drain_src = src.at[pl.ds(0, _CHUNK)]

            def process(j, buf):
                @plsc.parallel_loop(0, _CHUNK // _LANES, unroll=2)
                def scale(g):
                    evv = ev8_v[j, pl.ds(g * _LANES, _LANES)]
                    for lane in range(_LANES):
                        ev = evv[lane]
                        e = g * _LANES + lane
                        for jf in range(nvec):
                            sl = pl.ds(jf * _LANES, _LANES)
                            buf[e, sl] = ev * buf[e, sl]
                pltpu.sync_copy(buf, acc.at[row8_v.at[j]], add=True)

            def group(gi, carry):
                gb = eb + gi * grp
                pltpu.sync_copy(col2.at[pl.ds(gb, grp)], col8_v)
                pltpu.sync_copy(row2.at[pl.ds(gb, grp)], row8_v)
                pltpu.sync_copy(ev2.at[pl.ds(gb, grp)], ev8_v)

                @plsc.parallel_loop(0, grp, unroll=2)
                def adjj(j):
                    for m in range(_CHUNK // _LANES):
                        sl = pl.ds(m * _LANES, _LANES)
                        col8_v[j, sl] = col8_v[j, sl] + cn

                pltpu.async_copy(src.at[col8_v.at[0]], rows2.at[0], sems.at[0])
                pltpu.async_copy(src.at[col8_v.at[1]], rows2.at[1], sems.at[1])

                def chunk(j, c2):
                    par = jnp.bitwise_and(j, 1)
                    buf = rows2.at[par]
                    pltpu.make_async_copy(drain_src, buf, sems.at[par]).wait()
                    process(j, buf)
                    nj = j + 2
                    @pl.when(nj < grp)
                    def _():
                        pltpu.async_copy(src.at[col8_v.at[nj]], buf,
                                         sems.at[par])
                    return c2
                lax.fori_loop(0, grp, chunk, 0)
                return carry
            lax.fori_loop(0, ntg, group, 0)

        def writeback(l, write_h):
            g = gam_v[l, :]
            for k in range(nk):
                pltpu.sync_copy(acc.at[pl.ds(ra + k * _CHUNK, _CHUNK)],
                                stage_v)
                @pl.when(write_h)
                def _(_k=k):
                    pltpu.sync_copy(
                        stage_v, h.at[pl.ds(cn + ra + _k * _CHUNK, _CHUNK)])
                def wb(r, carry, _k=k):
                    for jf in range(nvec):
                        sl = pl.ds(jf * _LANES, _LANES)
                        out_acc[_k * _CHUNK + r, sl] = (
                            out_acc[_k * _CHUNK + r, sl] + g * stage_v[r, sl])
                    return carry
                lax.fori_loop(0, _CHUNK, wb, 0)

        zero_acc()
        plsc.subcore_barrier()
        edge_phase(xr)
        plsc.subcore_barrier()
        writeback(1, jnp.bool_(True))

        def hop(l, carry):
            zero_acc()
            plsc.subcore_barrier()
            edge_phase(h)
            plsc.subcore_barrier()
            writeback(l, l < _HOPS)
            return carry
        lax.fori_loop(2, _HOPS + 1, hop, 0)

        pltpu.sync_copy(out_acc, outr.at[pl.ds(cn + ra, rows_tile)])

    f32 = jnp.float32
    i32 = jnp.int32
    return pl.kernel(
        body,
        out_type=(
            jax.ShapeDtypeStruct((_NC * n_pad, dh), f32),
            jax.ShapeDtypeStruct((_NC * n_pad, dh), f32),
        ),
        mesh=mesh,
        compiler_params=pltpu.CompilerParams(use_tc_tiling_on_sc=False),
        scratch_types=[
            pltpu.VMEM((grp, _CHUNK), i32),
            pltpu.VMEM((grp, _CHUNK), i32),
            pltpu.VMEM((grp, _CHUNK), f32),
            pltpu.VMEM((2, _CHUNK, dh), f32),
            pltpu.VMEM((_CHUNK, dh), f32),
            pltpu.VMEM((_CHUNK, dh), f32),
            pltpu.VMEM((rows_tile, dh), f32),
            pltpu.VMEM((_LANES, _LANES), f32),
            pltpu.VMEM_SHARED((n_pad, dh), f32),
            pltpu.SemaphoreType.DMA((2,)),
        ],
    )


def kernel(X, edge_index, edge_values, gpr_weights):
    n, d = X.shape
    e = edge_values.shape[0]
    dh = d // _NC
    ep = -(-e // (_NS * _CHUNK * 8)) * (_NS * _CHUNK * 8)
    n_pad = -(-n // (_NS * _CHUNK)) * (_NS * _CHUNK)

    row = jnp.concatenate(
        [edge_index[0], jnp.zeros((ep - e,), jnp.int32)]).reshape(-1, _CHUNK)
    col = jnp.concatenate(
        [edge_index[1], jnp.zeros((ep - e,), jnp.int32)]).reshape(-1, _CHUNK)
    ev = jnp.concatenate(
        [edge_values, jnp.zeros((ep - e,), jnp.float32)]).reshape(-1, _CHUNK)
    xr = jnp.pad(X.reshape(n, _NC, dh).transpose(1, 0, 2),
                 ((0, 0), (0, n_pad - n), (0, 0))).reshape(_NC * n_pad, dh)
    gam = jnp.zeros((_LANES, _LANES), jnp.float32).at[
        :gpr_weights.shape[0]].set(gpr_weights[:, None])

    _, outr = _build(n, d, ep)(xr, col, row, ev, gam)
    return outr.reshape(_NC, n_pad, dh)[:, :n].transpose(1, 0, 2).reshape(n, d)


# speedup vs baseline: 4.5636x; 1.0921x over previous
import functools

import jax
import jax.numpy as jnp
from jax import lax
from jax.experimental import pallas as pl
from jax.experimental.pallas import tpu as pltpu
from jax.experimental.pallas import tpu_sc as plsc

_HOPS = 10
_NC = 2
_NS = 16
_LANES = 16
_CHUNK = 128


@functools.lru_cache(maxsize=None)
def _build(n, d, ep):
    dh = d // _NC
    nvec = dh // _LANES
    n_pad = -(-n // (_NS * _CHUNK)) * (_NS * _CHUNK)
    rows_tile = n_pad // _NS
    nk = rows_tile // _CHUNK
    cpt = ep // (_NS * _CHUNK)

    mesh = plsc.VectorSubcoreMesh(core_axis_name="c", subcore_axis_name="s")

    grp = 16
    ntg = cpt // grp

    def body(xr, col2, row2, ev2, gam, h, outr,
             col8_v, row8_v, ev8_v, rows4, stage_v, out_acc,
             gam_v, acc, gsem, ssem):
        c = lax.axis_index("c")
        s = lax.axis_index("s")
        cn = c * n_pad
        ra = s * rows_tile
        eb = s * cpt

        pltpu.sync_copy(gam, gam_v)
        zv = jnp.zeros((_LANES,), jnp.float32)

        def zero_stage():
            @plsc.parallel_loop(0, _CHUNK, unroll=4)
            def zs(r):
                for jf in range(nvec):
                    stage_v[r, pl.ds(jf * _LANES, _LANES)] = zv

        g0 = gam_v[0, :]
        for k in range(nk):
            pltpu.sync_copy(xr.at[pl.ds(cn + ra + k * _CHUNK, _CHUNK)],
                            stage_v)
            def initk(r, carry, _k=k):
                for jf in range(nvec):
                    sl = pl.ds(jf * _LANES, _LANES)
                    out_acc[_k * _CHUNK + r, sl] = g0 * stage_v[r, sl]
                return carry
            lax.fori_loop(0, _CHUNK, initk, 0)

        def zero_acc():
            for k in range(nk):
                pltpu.sync_copy(stage_v, acc.at[pl.ds(ra + k * _CHUNK, _CHUNK)])

        def edge_phase(src):
            drain_src = src.at[pl.ds(0, _CHUNK)]

            def process(j, buf):
                @plsc.parallel_loop(0, _CHUNK // _LANES, unroll=2)
                def scale(g):
                    evv = ev8_v[j, pl.ds(g * _LANES, _LANES)]
                    for lane in range(_LANES):
                        ev = evv[lane]
                        e = g * _LANES + lane
                        for jf in range(nvec):
                            sl = pl.ds(jf * _LANES, _LANES)
                            buf[e, sl] = ev * buf[e, sl]
                pltpu.async_copy(buf, acc.at[row8_v.at[j]],
                                 ssem.at[jnp.bitwise_and(j, 3)], add=True)

            def group(gi, carry):
                gb = eb + gi * grp
                pltpu.sync_copy(col2.at[pl.ds(gb, grp)], col8_v)
                pltpu.sync_copy(row2.at[pl.ds(gb, grp)], row8_v)
                pltpu.sync_copy(ev2.at[pl.ds(gb, grp)], ev8_v)

                @plsc.parallel_loop(0, grp, unroll=2)
                def adjj(j):
                    for m in range(_CHUNK // _LANES):
                        sl = pl.ds(m * _LANES, _LANES)
                        col8_v[j, sl] = col8_v[j, sl] + cn

                pltpu.async_copy(src.at[col8_v.at[0]], rows4.at[0], gsem.at[0])
                pltpu.async_copy(src.at[col8_v.at[1]], rows4.at[1], gsem.at[1])

                def chunk(j, c2):
                    s = jnp.bitwise_and(j, 3)
                    nj = j + 2
                    s2 = jnp.bitwise_and(nj, 3)
                    @pl.when(jnp.logical_and(j >= 2, nj < grp))
                    def _():
                        pltpu.make_async_copy(
                            drain_src, rows4.at[s2], ssem.at[s2]).wait()
                    @pl.when(nj < grp)
                    def _():
                        pltpu.async_copy(src.at[col8_v.at[nj]], rows4.at[s2],
                                         gsem.at[s2])
                    buf = rows4.at[s]
                    pltpu.make_async_copy(drain_src, buf, gsem.at[s]).wait()
                    process(j, buf)
                    return c2
                lax.fori_loop(0, grp, chunk, 0)

                for s in range(4):
                    pltpu.make_async_copy(
                        drain_src, rows4.at[s], ssem.at[s]).wait()
                return carry
            lax.fori_loop(0, ntg, group, 0)

        def writeback(l, write_h):
            g = gam_v[l, :]
            for k in range(nk):
                pltpu.sync_copy(acc.at[pl.ds(ra + k * _CHUNK, _CHUNK)],
                                stage_v)
                @pl.when(write_h)
                def _(_k=k):
                    pltpu.sync_copy(
                        stage_v, h.at[pl.ds(cn + ra + _k * _CHUNK, _CHUNK)])
                def wb(r, carry, _k=k):
                    for jf in range(nvec):
                        sl = pl.ds(jf * _LANES, _LANES)
                        out_acc[_k * _CHUNK + r, sl] = (
                            out_acc[_k * _CHUNK + r, sl] + g * stage_v[r, sl])
                    return carry
                lax.fori_loop(0, _CHUNK, wb, 0)

        zero_stage()
        zero_acc()
        plsc.subcore_barrier()
        edge_phase(xr)
        plsc.subcore_barrier()
        writeback(1, jnp.bool_(True))

        def hop(l, carry):
            zero_stage()
            zero_acc()
            plsc.subcore_barrier()
            edge_phase(h)
            plsc.subcore_barrier()
            writeback(l, l < _HOPS)
            return carry
        lax.fori_loop(2, _HOPS + 1, hop, 0)

        pltpu.sync_copy(out_acc, outr.at[pl.ds(cn + ra, rows_tile)])

    f32 = jnp.float32
    i32 = jnp.int32
    return pl.kernel(
        body,
        out_type=(
            jax.ShapeDtypeStruct((_NC * n_pad, dh), f32),
            jax.ShapeDtypeStruct((_NC * n_pad, dh), f32),
        ),
        mesh=mesh,
        compiler_params=pltpu.CompilerParams(use_tc_tiling_on_sc=False),
        scratch_types=[
            pltpu.VMEM((grp, _CHUNK), i32),
            pltpu.VMEM((grp, _CHUNK), i32),
            pltpu.VMEM((grp, _CHUNK), f32),
            pltpu.VMEM((4, _CHUNK, dh), f32),
            pltpu.VMEM((_CHUNK, dh), f32),
            pltpu.VMEM((rows_tile, dh), f32),
            pltpu.VMEM((_LANES, _LANES), f32),
            pltpu.VMEM_SHARED((n_pad, dh), f32),
            pltpu.SemaphoreType.DMA((4,)),
            pltpu.SemaphoreType.DMA((4,)),
        ],
    )


def kernel(X, edge_index, edge_values, gpr_weights):
    n, d = X.shape
    e = edge_values.shape[0]
    dh = d // _NC
    ep = -(-e // (_NS * _CHUNK * 8)) * (_NS * _CHUNK * 8)
    n_pad = -(-n // (_NS * _CHUNK)) * (_NS * _CHUNK)

    row = jnp.concatenate(
        [edge_index[0], jnp.zeros((ep - e,), jnp.int32)]).reshape(-1, _CHUNK)
    col = jnp.concatenate(
        [edge_index[1], jnp.zeros((ep - e,), jnp.int32)]).reshape(-1, _CHUNK)
    ev = jnp.concatenate(
        [edge_values, jnp.zeros((ep - e,), jnp.float32)]).reshape(-1, _CHUNK)
    xr = jnp.pad(X.reshape(n, _NC, dh).transpose(1, 0, 2),
                 ((0, 0), (0, n_pad - n), (0, 0))).reshape(_NC * n_pad, dh)
    gam = jnp.zeros((_LANES, _LANES), jnp.float32).at[
        :gpr_weights.shape[0]].set(gpr_weights[:, None])

    _, outr = _build(n, d, ep)(xr, col, row, ev, gam)
    return outr.reshape(_NC, n_pad, dh)[:, :n].transpose(1, 0, 2).reshape(n, d)


# speedup vs baseline: 4.5661x; 1.0005x over previous
import functools

import jax
import jax.numpy as jnp
from jax import lax
from jax.experimental import pallas as pl
from jax.experimental.pallas import tpu as pltpu
from jax.experimental.pallas import tpu_sc as plsc

_HOPS = 10
_DNUMS = jax.lax.GatherDimensionNumbers(
    offset_dims=(), collapsed_slice_dims=(0,), start_index_map=(0,))


def _splat(v, lane):
    idx = jnp.full((16, 1), lane, jnp.int32)
    return jax.lax.gather(v, idx, _DNUMS, (1,),
                          mode=jax.lax.GatherScatterMode.PROMISE_IN_BOUNDS)

_NC = 2
_NS = 16
_LANES = 16
_CHUNK = 128


@functools.lru_cache(maxsize=None)
def _build(n, d, ep):
    dh = d // _NC
    nvec = dh // _LANES
    n_pad = -(-n // (_NS * _CHUNK)) * (_NS * _CHUNK)
    rows_tile = n_pad // _NS
    nk = rows_tile // _CHUNK
    cpt = ep // (_NS * _CHUNK)

    mesh = plsc.VectorSubcoreMesh(core_axis_name="c", subcore_axis_name="s")

    grp = 16
    ntg = cpt // grp

    def body(xr, col2, row2, ev2, gam, h, outr,
             col8_v, row8_v, ev8_v, rows4, stage_v, out_acc,
             gam_v, acc, gsem, ssem):
        c = lax.axis_index("c")
        s = lax.axis_index("s")
        cn = c * n_pad
        ra = s * rows_tile
        eb = s * cpt

        pltpu.sync_copy(gam, gam_v)
        zv = jnp.zeros((_LANES,), jnp.float32)

        def zero_stage():
            @plsc.parallel_loop(0, _CHUNK, unroll=4)
            def zs(r):
                for jf in range(nvec):
                    stage_v[r, pl.ds(jf * _LANES, _LANES)] = zv

        g0 = gam_v[0, :]
        for k in range(nk):
            pltpu.sync_copy(xr.at[pl.ds(cn + ra + k * _CHUNK, _CHUNK)],
                            stage_v)
            def initk(r, carry, _k=k):
                for jf in range(nvec):
                    sl = pl.ds(jf * _LANES, _LANES)
                    out_acc[_k * _CHUNK + r, sl] = g0 * stage_v[r, sl]
                return carry
            lax.fori_loop(0, _CHUNK, initk, 0)

        def zero_acc():
            for k in range(nk):
                pltpu.sync_copy(stage_v, acc.at[pl.ds(ra + k * _CHUNK, _CHUNK)])

        def edge_phase(src):
            drain_src = src.at[pl.ds(0, _CHUNK)]

            def process(j, buf):
                @plsc.parallel_loop(0, _CHUNK // _LANES, unroll=2)
                def scale(g):
                    evv = ev8_v[j, pl.ds(g * _LANES, _LANES)]
                    for lane in range(_LANES):
                        evb = _splat(evv, lane)
                        e = g * _LANES + lane
                        for jf in range(nvec):
                            sl = pl.ds(jf * _LANES, _LANES)
                            buf[e, sl] = evb * buf[e, sl]
                pltpu.async_copy(buf, acc.at[row8_v.at[j]],
                                 ssem.at[jnp.bitwise_and(j, 3)], add=True)

            def group(gi, carry):
                gb = eb + gi * grp
                pltpu.sync_copy(col2.at[pl.ds(gb, grp)], col8_v)
                pltpu.sync_copy(row2.at[pl.ds(gb, grp)], row8_v)
                pltpu.sync_copy(ev2.at[pl.ds(gb, grp)], ev8_v)

                @plsc.parallel_loop(0, grp, unroll=2)
                def adjj(j):
                    for m in range(_CHUNK // _LANES):
                        sl = pl.ds(m * _LANES, _LANES)
                        col8_v[j, sl] = col8_v[j, sl] + cn

                pltpu.async_copy(src.at[col8_v.at[0]], rows4.at[0], gsem.at[0])
                pltpu.async_copy(src.at[col8_v.at[1]], rows4.at[1], gsem.at[1])

                def chunk(j, c2):
                    s = jnp.bitwise_and(j, 3)
                    nj = j + 2
                    s2 = jnp.bitwise_and(nj, 3)
                    @pl.when(jnp.logical_and(j >= 2, nj < grp))
                    def _():
                        pltpu.make_async_copy(
                            drain_src, rows4.at[s2], ssem.at[s2]).wait()
                    @pl.when(nj < grp)
                    def _():
                        pltpu.async_copy(src.at[col8_v.at[nj]], rows4.at[s2],
                                         gsem.at[s2])
                    buf = rows4.at[s]
                    pltpu.make_async_copy(drain_src, buf, gsem.at[s]).wait()
                    process(j, buf)
                    return c2
                lax.fori_loop(0, grp, chunk, 0)

                for s in range(4):
                    pltpu.make_async_copy(
                        drain_src, rows4.at[s], ssem.at[s]).wait()
                return carry
            lax.fori_loop(0, ntg, group, 0)

        def writeback(l, write_h):
            g = gam_v[l, :]
            for k in range(nk):
                pltpu.sync_copy(acc.at[pl.ds(ra + k * _CHUNK, _CHUNK)],
                                stage_v)
                @pl.when(write_h)
                def _(_k=k):
                    pltpu.sync_copy(
                        stage_v, h.at[pl.ds(cn + ra + _k * _CHUNK, _CHUNK)])
                def wb(r, carry, _k=k):
                    for jf in range(nvec):
                        sl = pl.ds(jf * _LANES, _LANES)
                        out_acc[_k * _CHUNK + r, sl] = (
                            out_acc[_k * _CHUNK + r, sl] + g * stage_v[r, sl])
                    return carry
                lax.fori_loop(0, _CHUNK, wb, 0)

        zero_stage()
        zero_acc()
        plsc.subcore_barrier()
        edge_phase(xr)
        plsc.subcore_barrier()
        writeback(1, jnp.bool_(True))

        def hop(l, carry):
            zero_stage()
            zero_acc()
            plsc.subcore_barrier()
            edge_phase(h)
            plsc.subcore_barrier()
            writeback(l, l < _HOPS)
            return carry
        lax.fori_loop(2, _HOPS + 1, hop, 0)

        pltpu.sync_copy(out_acc, outr.at[pl.ds(cn + ra, rows_tile)])

    f32 = jnp.float32
    i32 = jnp.int32
    return pl.kernel(
        body,
        out_type=(
            jax.ShapeDtypeStruct((_NC * n_pad, dh), f32),
            jax.ShapeDtypeStruct((_NC * n_pad, dh), f32),
        ),
        mesh=mesh,
        compiler_params=pltpu.CompilerParams(use_tc_tiling_on_sc=False),
        scratch_types=[
            pltpu.VMEM((grp, _CHUNK), i32),
            pltpu.VMEM((grp, _CHUNK), i32),
            pltpu.VMEM((grp, _CHUNK), f32),
            pltpu.VMEM((4, _CHUNK, dh), f32),
            pltpu.VMEM((_CHUNK, dh), f32),
            pltpu.VMEM((rows_tile, dh), f32),
            pltpu.VMEM((_LANES, _LANES), f32),
            pltpu.VMEM_SHARED((n_pad, dh), f32),
            pltpu.SemaphoreType.DMA((4,)),
            pltpu.SemaphoreType.DMA((4,)),
        ],
    )


def kernel(X, edge_index, edge_values, gpr_weights):
    n, d = X.shape
    e = edge_values.shape[0]
    dh = d // _NC
    ep = -(-e // (_NS * _CHUNK * 8)) * (_NS * _CHUNK * 8)
    n_pad = -(-n // (_NS * _CHUNK)) * (_NS * _CHUNK)

    row = jnp.concatenate(
        [edge_index[0], jnp.zeros((ep - e,), jnp.int32)]).reshape(-1, _CHUNK)
    col = jnp.concatenate(
        [edge_index[1], jnp.zeros((ep - e,), jnp.int32)]).reshape(-1, _CHUNK)
    ev = jnp.concatenate(
        [edge_values, jnp.zeros((ep - e,), jnp.float32)]).reshape(-1, _CHUNK)
    xr = jnp.pad(X.reshape(n, _NC, dh).transpose(1, 0, 2),
                 ((0, 0), (0, n_pad - n), (0, 0))).reshape(_NC * n_pad, dh)
    gam = jnp.zeros((_LANES, _LANES), jnp.float32).at[
        :gpr_weights.shape[0]].set(gpr_weights[:, None])

    _, outr = _build(n, d, ep)(xr, col, row, ev, gam)
    return outr.reshape(_NC, n_pad, dh)[:, :n].transpose(1, 0, 2).reshape(n, d)
